# baseline (speedup 1.0000x reference)
import functools

import jax
import jax.numpy as jnp
from jax import lax
from jax.experimental import pallas as pl
from jax.experimental.pallas import tpu as pltpu
from jax.experimental.pallas import tpu_sc as plsc

N = 10000
E = 320000
IN_DIM = 128
HID = 32
NCLS = 2

NC = 2
NS = 16
NW = NC * NS
CHUNK = 128
NBUF = 4
NCH = 80
NOUT = NCH // NBUF
NCHA = NCH + NBUF
E_PAD = NW * NCH * CHUNK
NP = 10112
RPT = NP // NS
DEG_W = 16
DEG_KB = 8

_mesh = plsc.VectorSubcoreMesh(core_axis_name="c", subcore_axis_name="s")
_sc_params = pltpu.CompilerParams(use_tc_tiling_on_sc=False)


def _worker_id():
    return lax.axis_index("s") * NC + lax.axis_index("c")


@functools.partial(
    pl.kernel,
    out_type=jax.ShapeDtypeStruct((NC, NP, DEG_W), jnp.float32),
    mesh=_mesh,
    compiler_params=_sc_params,
    scratch_types=[
        pltpu.VMEM_SHARED((NP, DEG_W), jnp.float32),
        pltpu.VMEM((CHUNK, DEG_W), jnp.float32),
        pltpu.VMEM((NCHA, CHUNK), jnp.int32),
        pltpu.SemaphoreType.DMA,
    ],
)
def _sc_degree(dst3, ones_hbm, zeros_hbm, out, acc, ones_v, idx_d, sem):
    cid = lax.axis_index("c")
    sid = lax.axis_index("s")
    wid = _worker_id()
    base = sid * RPT
    pltpu.sync_copy(dst3.at[wid], idx_d)
    pltpu.sync_copy(ones_hbm, ones_v)
    pltpu.sync_copy(zeros_hbm.at[pl.ds(base, RPT)], acc.at[pl.ds(base, RPT)])
    plsc.subcore_barrier()

    def body(j0, carry):
        descs = [
            pltpu.async_copy(ones_v, acc.at[idx_d.at[j0 * DEG_KB + b]], sem,
                             add=True)
            for b in range(DEG_KB)
        ]
        for d in descs:
            d.wait()
        return carry

    lax.fori_loop(0, NCH // DEG_KB, body, 0)
    plsc.subcore_barrier()
    pltpu.sync_copy(acc.at[pl.ds(base, RPT)], out.at[cid, pl.ds(base, RPT)])


@functools.partial(
    pl.kernel,
    out_type=jax.ShapeDtypeStruct((NC, NP, HID), jnp.float32),
    mesh=_mesh,
    compiler_params=_sc_params,
    scratch_types=[
        pltpu.VMEM_SHARED((NP, HID), jnp.float32),
        [pltpu.VMEM((CHUNK, HID), jnp.float32) for _ in range(NBUF)],
        pltpu.VMEM((NCHA, CHUNK), jnp.int32),
        pltpu.VMEM((NCHA, CHUNK), jnp.int32),
        [pltpu.SemaphoreType.DMA for _ in range(NBUF)],
        [pltpu.SemaphoreType.DMA for _ in range(NBUF)],
    ],
)
def _sc_spmm(xp, src3, dst3, zeros_hbm, out, acc, rows, idx_s, idx_d,
             gsems, ssems):
    cid = lax.axis_index("c")
    sid = lax.axis_index("s")
    wid = _worker_id()
    base = sid * RPT
    pltpu.sync_copy(src3.at[wid], idx_s)
    pltpu.sync_copy(dst3.at[wid], idx_d)
    for b in range(NBUF):
        pltpu.async_copy(xp.at[idx_s.at[b]], rows[b], gsems[b])
    pltpu.sync_copy(zeros_hbm.at[pl.ds(base, RPT)], acc.at[pl.ds(base, RPT)])
    plsc.subcore_barrier()

    def body(j0, carry):
        sdescs = []
        for b in range(NBUF):
            j = j0 * NBUF + b
            pltpu.make_async_copy(xp.at[idx_s.at[j]], rows[b],
                                  gsems[b]).wait()
            sdescs.append(
                pltpu.async_copy(rows[b], acc.at[idx_d.at[j]], ssems[b],
                                 add=True))
        for b in range(NBUF):
            jn = (j0 + 1) * NBUF + b
            sdescs[b].wait()
            pltpu.async_copy(xp.at[idx_s.at[jn]], rows[b], gsems[b])
        return carry

    lax.fori_loop(0, NOUT, body, 0)
    for b in range(NBUF):
        pltpu.make_async_copy(xp.at[idx_s.at[NCH + b]], rows[b],
                              gsems[b]).wait()
    plsc.subcore_barrier()
    pltpu.sync_copy(acc.at[pl.ds(base, RPT)], out.at[cid, pl.ds(base, RPT)])


@functools.partial(
    pl.kernel,
    out_type=(
        jax.ShapeDtypeStruct((NW, NCH * CHUNK, HID), jnp.float32),
        jax.ShapeDtypeStruct((NW, NCH * CHUNK, HID), jnp.float32),
    ),
    mesh=_mesh,
    compiler_params=_sc_params,
    scratch_types=[
        [pltpu.VMEM((CHUNK, HID), jnp.float32) for _ in range(NBUF)],
        [pltpu.VMEM((CHUNK, HID), jnp.float32) for _ in range(NBUF)],
        pltpu.VMEM((NCHA, CHUNK), jnp.int32),
        pltpu.VMEM((NCHA, CHUNK), jnp.int32),
        [pltpu.SemaphoreType.DMA for _ in range(NBUF)],
        [pltpu.SemaphoreType.DMA for _ in range(NBUF)],
        [pltpu.SemaphoreType.DMA for _ in range(NBUF)],
        [pltpu.SemaphoreType.DMA for _ in range(NBUF)],
    ],
)
def _sc_edge_gather(h3, src3, dst3, gs, gd, rows_s, rows_d, idx_s, idx_d,
                    gsems_s, gsems_d, wsems_s, wsems_d):
    wid = _worker_id()
    pltpu.sync_copy(src3.at[wid], idx_s)
    pltpu.sync_copy(dst3.at[wid], idx_d)
    for b in range(NBUF):
        pltpu.async_copy(h3.at[idx_s.at[b]], rows_s[b], gsems_s[b])
        pltpu.async_copy(h3.at[idx_d.at[b]], rows_d[b], gsems_d[b])

    def body(j0, carry):
        wdescs = []
        for b in range(NBUF):
            j = j0 * NBUF + b
            pltpu.make_async_copy(h3.at[idx_s.at[j]], rows_s[b],
                                  gsems_s[b]).wait()
            wdescs.append(
                pltpu.async_copy(rows_s[b],
                                 gs.at[wid, pl.ds(j * CHUNK, CHUNK)],
                                 wsems_s[b]))
            pltpu.make_async_copy(h3.at[idx_d.at[j]], rows_d[b],
                                  gsems_d[b]).wait()
            wdescs.append(
                pltpu.async_copy(rows_d[b],
                                 gd.at[wid, pl.ds(j * CHUNK, CHUNK)],
                                 wsems_d[b]))
        for b in range(NBUF):
            jn = (j0 + 1) * NBUF + b
            wdescs[2 * b].wait()
            wdescs[2 * b + 1].wait()
            pltpu.async_copy(h3.at[idx_s.at[jn]], rows_s[b], gsems_s[b])
            pltpu.async_copy(h3.at[idx_d.at[jn]], rows_d[b], gsems_d[b])
        return carry

    lax.fori_loop(0, NOUT, body, 0)
    for b in range(NBUF):
        pltpu.make_async_copy(h3.at[idx_s.at[NCH + b]], rows_s[b],
                              gsems_s[b]).wait()
        pltpu.make_async_copy(h3.at[idx_d.at[NCH + b]], rows_d[b],
                              gsems_d[b]).wait()


_BLK = 2528
_EPS = 1e-5


def _prep_body(nf, w1t, d0, d1, tp, dv):
    deg = d0[...] + d1[...] + 1.0
    di = lax.rsqrt(deg)
    t = jnp.dot(nf[...], w1t[...], preferred_element_type=jnp.float32)
    tp[...] = di * t
    dv[...] = di


def _tc_prep(nf_p, w1t, d0, d1):
    return pl.pallas_call(
        _prep_body,
        grid=(NP // _BLK,),
        in_specs=[
            pl.BlockSpec((_BLK, IN_DIM), lambda i: (i, 0)),
            pl.BlockSpec((IN_DIM, HID), lambda i: (0, 0)),
            pl.BlockSpec((_BLK, 1), lambda i: (i, 0)),
            pl.BlockSpec((_BLK, 1), lambda i: (i, 0)),
        ],
        out_specs=[
            pl.BlockSpec((_BLK, HID), lambda i: (i, 0)),
            pl.BlockSpec((_BLK, 1), lambda i: (i, 0)),
        ],
        out_shape=[
            jax.ShapeDtypeStruct((NP, HID), jnp.float32),
            jax.ShapeDtypeStruct((NP, 1), jnp.float32),
        ],
    )(nf_p, w1t, d0, d1)


def _layer_body(z0, z1, tp, dv, b, g, be, wnt, out):
    di = dv[...]
    s = di * (z0[...] + z1[...] + tp[...]) + b[...]
    mu = jnp.mean(s, axis=-1, keepdims=True)
    var = jnp.mean((s - mu) ** 2, axis=-1, keepdims=True)
    h = jnp.maximum((s - mu) * lax.rsqrt(var + _EPS) * g[...] + be[...], 0.0)
    if wnt is not None:
        out[...] = di * jnp.dot(h, wnt[...], preferred_element_type=jnp.float32)
    else:
        out[...] = h


def _tc_layer(z0, z1, tp, dv, b, g, be, wnt):
    hid_spec = pl.BlockSpec((_BLK, HID), lambda i: (i, 0))
    vec_spec = pl.BlockSpec((1, HID), lambda i: (0, 0))
    in_specs = [hid_spec, hid_spec, hid_spec,
                pl.BlockSpec((_BLK, 1), lambda i: (i, 0)),
                vec_spec, vec_spec, vec_spec]
    args = [z0, z1, tp, dv, b, g, be]
    if wnt is not None:
        body = _layer_body
        in_specs.append(pl.BlockSpec((HID, HID), lambda i: (0, 0)))
        args.append(wnt)
    else:
        def body(z0, z1, tp, dv, b, g, be, out):
            _layer_body(z0, z1, tp, dv, b, g, be, None, out)
    return pl.pallas_call(
        body,
        grid=(NP // _BLK,),
        in_specs=in_specs,
        out_specs=hid_spec,
        out_shape=jax.ShapeDtypeStruct((NP, HID), jnp.float32),
    )(*args)


_EBLK = 4096


def _head_body(gs, gd, cwt, cb, out):
    out[...] = (jnp.dot(gs[...] * gd[...], cwt[...],
                        preferred_element_type=jnp.float32) + cb[...])


def _tc_head(gs, gd, cwt, cb):
    return pl.pallas_call(
        _head_body,
        grid=(E_PAD // _EBLK,),
        in_specs=[
            pl.BlockSpec((_EBLK, HID), lambda i: (i, 0)),
            pl.BlockSpec((_EBLK, HID), lambda i: (i, 0)),
            pl.BlockSpec((HID, NCLS), lambda i: (0, 0)),
            pl.BlockSpec((1, NCLS), lambda i: (0, 0)),
        ],
        out_specs=pl.BlockSpec((_EBLK, NCLS), lambda i: (i, 0)),
        out_shape=jax.ShapeDtypeStruct((E_PAD, NCLS), jnp.float32),
    )(gs, gd, cwt, cb)


def kernel(node_feat, edge_index, fc1_w, fc1_b, fc2_w, fc2_b, fc3_w, fc3_b,
           ln1_g, ln1_b, ln2_g, ln2_b, ln3_g, ln3_b, cls_w, cls_b):
    ei = edge_index.astype(jnp.int32)
    src = jnp.concatenate(
        [ei[0], jnp.zeros((E_PAD - E,), jnp.int32)]).reshape(NW, NCH, CHUNK)
    src = jnp.concatenate(
        [src, jnp.zeros((NW, NBUF, CHUNK), jnp.int32)], axis=1)
    dst = jnp.concatenate(
        [ei[1], jnp.full((E_PAD - E,), N, jnp.int32)]).reshape(NW, NCH, CHUNK)
    dst = jnp.concatenate(
        [dst, jnp.full((NW, NBUF, CHUNK), N, jnp.int32)], axis=1)

    nf_p = jnp.pad(node_feat, ((0, NP - N), (0, 0)))
    zeros_deg = jnp.zeros((NP, DEG_W), jnp.float32)
    ones_deg = jnp.ones((CHUNK, DEG_W), jnp.float32)
    zeros_hid = jnp.zeros((NP, HID), jnp.float32)

    degp = _sc_degree(dst, ones_deg, zeros_deg)
    d0 = degp[0, :, :1]
    d1 = degp[1, :, :1]

    t1p, dv = _tc_prep(nf_p, fc1_w.T, d0, d1)
    z = _sc_spmm(t1p, src, dst, zeros_hid)
    t2p = _tc_layer(z[0], z[1], t1p, dv,
                    fc1_b.reshape(1, HID), ln1_g.reshape(1, HID),
                    ln1_b.reshape(1, HID), fc2_w.T)
    z = _sc_spmm(t2p, src, dst, zeros_hid)
    t3p = _tc_layer(z[0], z[1], t2p, dv,
                    fc2_b.reshape(1, HID), ln2_g.reshape(1, HID),
                    ln2_b.reshape(1, HID), fc3_w.T)
    z = _sc_spmm(t3p, src, dst, zeros_hid)
    h3 = _tc_layer(z[0], z[1], t3p, dv,
                   fc3_b.reshape(1, HID), ln3_g.reshape(1, HID),
                   ln3_b.reshape(1, HID), None)

    gs, gd = _sc_edge_gather(h3, src, dst)
    logits = _tc_head(gs.reshape(E_PAD, HID), gd.reshape(E_PAD, HID),
                      cls_w.T, cls_b.reshape(1, NCLS))
    return logits[:E]


# speedup vs baseline: 1.4214x; 1.4214x over previous
import functools

import jax
import jax.numpy as jnp
from jax import lax
from jax.experimental import pallas as pl
from jax.experimental.pallas import tpu as pltpu
from jax.experimental.pallas import tpu_sc as plsc

N = 10000
E = 320000
IN_DIM = 128
HID = 32
NCLS = 2

NC = 2
NS = 16
NW = NC * NS
CHUNK = 128
GK = 4
NCH = 80
E_PAD = NW * NCH * CHUNK
NP = 10112
RPT = NP // NS
DEG_W = 16
DEG_KB = 8

_mesh = plsc.VectorSubcoreMesh(core_axis_name="c", subcore_axis_name="s")
_sc_params = pltpu.CompilerParams(use_tc_tiling_on_sc=False)


def _worker_id():
    return lax.axis_index("s") * NC + lax.axis_index("c")


@functools.partial(
    pl.kernel,
    out_type=jax.ShapeDtypeStruct((NC, NP, DEG_W), jnp.float32),
    mesh=_mesh,
    compiler_params=_sc_params,
    scratch_types=[
        pltpu.VMEM_SHARED((NP, DEG_W), jnp.float32),
        pltpu.VMEM((CHUNK, DEG_W), jnp.float32),
        pltpu.VMEM((NCH, CHUNK), jnp.int32),
        pltpu.SemaphoreType.DMA,
    ],
)
def _sc_degree(dst3, ones_hbm, zeros_hbm, out, acc, ones_v, idx_d, sem):
    cid = lax.axis_index("c")
    sid = lax.axis_index("s")
    wid = _worker_id()
    base = sid * RPT
    pltpu.sync_copy(dst3.at[wid], idx_d)
    pltpu.sync_copy(ones_hbm, ones_v)
    pltpu.sync_copy(zeros_hbm.at[pl.ds(base, RPT)], acc.at[pl.ds(base, RPT)])
    plsc.subcore_barrier()

    def body(j0, carry):
        descs = [
            pltpu.async_copy(ones_v, acc.at[idx_d.at[j0 * DEG_KB + b]], sem,
                             add=True)
            for b in range(DEG_KB)
        ]
        for d in descs:
            d.wait()
        return carry

    lax.fori_loop(0, NCH // DEG_KB, body, 0)
    plsc.subcore_barrier()
    pltpu.sync_copy(acc.at[pl.ds(base, RPT)], out.at[cid, pl.ds(base, RPT)])


@functools.partial(
    pl.kernel,
    out_type=jax.ShapeDtypeStruct((NC, NP, HID), jnp.float32),
    mesh=_mesh,
    compiler_params=_sc_params,
    scratch_types=[
        pltpu.VMEM_SHARED((NP, HID), jnp.float32),
        [pltpu.VMEM((CHUNK, HID), jnp.float32) for _ in range(2 * GK)],
        pltpu.VMEM((NCH, CHUNK), jnp.int32),
        pltpu.VMEM((NCH, CHUNK), jnp.int32),
        pltpu.SemaphoreType.DMA,
        pltpu.SemaphoreType.DMA,
    ],
)
def _sc_spmm(xp, src3, dst3, zeros_hbm, out, acc, rows, idx_s, idx_d,
             gsem, ssem):
    cid = lax.axis_index("c")
    sid = lax.axis_index("s")
    wid = _worker_id()
    base = sid * RPT
    pltpu.sync_copy(src3.at[wid], idx_s)
    pltpu.sync_copy(dst3.at[wid], idx_d)
    pltpu.sync_copy(zeros_hbm.at[pl.ds(base, RPT)], acc.at[pl.ds(base, RPT)])
    plsc.subcore_barrier()

    def body(j0, carry):
        c0 = j0 * 2 * GK
        ga = [pltpu.async_copy(xp.at[idx_s.at[c0 + b]], rows[b], gsem)
              for b in range(GK)]
        for d in ga:
            d.wait()
        sa = [pltpu.async_copy(rows[b], acc.at[idx_d.at[c0 + b]], ssem,
                               add=True) for b in range(GK)]
        gb = [pltpu.async_copy(xp.at[idx_s.at[c0 + GK + b]], rows[GK + b],
                               gsem) for b in range(GK)]
        for d in gb:
            d.wait()
        sb = [pltpu.async_copy(rows[GK + b], acc.at[idx_d.at[c0 + GK + b]],
                               ssem, add=True) for b in range(GK)]
        for d in sa + sb:
            d.wait()
        return carry

    lax.fori_loop(0, NCH // (2 * GK), body, 0)
    plsc.subcore_barrier()
    pltpu.sync_copy(acc.at[pl.ds(base, RPT)], out.at[cid, pl.ds(base, RPT)])


@functools.partial(
    pl.kernel,
    out_type=(
        jax.ShapeDtypeStruct((NW, NCH * CHUNK, HID), jnp.float32),
        jax.ShapeDtypeStruct((NW, NCH * CHUNK, HID), jnp.float32),
    ),
    mesh=_mesh,
    compiler_params=_sc_params,
    scratch_types=[
        [pltpu.VMEM((CHUNK, HID), jnp.float32) for _ in range(2 * GK)],
        [pltpu.VMEM((CHUNK, HID), jnp.float32) for _ in range(2 * GK)],
        pltpu.VMEM((NCH, CHUNK), jnp.int32),
        pltpu.VMEM((NCH, CHUNK), jnp.int32),
        pltpu.SemaphoreType.DMA,
        pltpu.SemaphoreType.DMA,
    ],
)
def _sc_edge_gather(h3, src3, dst3, gs, gd, rows_s, rows_d, idx_s, idx_d,
                    gsem, wsem):
    wid = _worker_id()
    pltpu.sync_copy(src3.at[wid], idx_s)
    pltpu.sync_copy(dst3.at[wid], idx_d)

    def grp_gather(c0, lo):
        descs = []
        for b in range(GK):
            descs.append(pltpu.async_copy(h3.at[idx_s.at[c0 + b]],
                                          rows_s[lo + b], gsem))
            descs.append(pltpu.async_copy(h3.at[idx_d.at[c0 + b]],
                                          rows_d[lo + b], gsem))
        return descs

    def grp_write(c0, lo):
        descs = []
        for b in range(GK):
            j = c0 + b
            descs.append(pltpu.async_copy(
                rows_s[lo + b], gs.at[wid, pl.ds(j * CHUNK, CHUNK)], wsem))
            descs.append(pltpu.async_copy(
                rows_d[lo + b], gd.at[wid, pl.ds(j * CHUNK, CHUNK)], wsem))
        return descs

    def body(j0, carry):
        c0 = j0 * 2 * GK
        ga = grp_gather(c0, 0)
        for d in ga:
            d.wait()
        wa = grp_write(c0, 0)
        gb = grp_gather(c0 + GK, GK)
        for d in gb:
            d.wait()
        wb = grp_write(c0 + GK, GK)
        for d in wa + wb:
            d.wait()
        return carry

    lax.fori_loop(0, NCH // (2 * GK), body, 0)


_BLK = 2528
_EPS = 1e-5


def _prep_body(nf, w1t, d0, d1, tp, dv):
    deg = d0[...] + d1[...] + 1.0
    di = lax.rsqrt(deg)
    t = jnp.dot(nf[...], w1t[...], preferred_element_type=jnp.float32)
    tp[...] = di * t
    dv[...] = di


def _tc_prep(nf_p, w1t, d0, d1):
    return pl.pallas_call(
        _prep_body,
        grid=(NP // _BLK,),
        in_specs=[
            pl.BlockSpec((_BLK, IN_DIM), lambda i: (i, 0)),
            pl.BlockSpec((IN_DIM, HID), lambda i: (0, 0)),
            pl.BlockSpec((_BLK, 1), lambda i: (i, 0)),
            pl.BlockSpec((_BLK, 1), lambda i: (i, 0)),
        ],
        out_specs=[
            pl.BlockSpec((_BLK, HID), lambda i: (i, 0)),
            pl.BlockSpec((_BLK, 1), lambda i: (i, 0)),
        ],
        out_shape=[
            jax.ShapeDtypeStruct((NP, HID), jnp.float32),
            jax.ShapeDtypeStruct((NP, 1), jnp.float32),
        ],
    )(nf_p, w1t, d0, d1)


def _layer_body(z0, z1, tp, dv, b, g, be, wnt, out):
    di = dv[...]
    s = di * (z0[...] + z1[...] + tp[...]) + b[...]
    mu = jnp.mean(s, axis=-1, keepdims=True)
    var = jnp.mean((s - mu) ** 2, axis=-1, keepdims=True)
    h = jnp.maximum((s - mu) * lax.rsqrt(var + _EPS) * g[...] + be[...], 0.0)
    if wnt is not None:
        out[...] = di * jnp.dot(h, wnt[...], preferred_element_type=jnp.float32)
    else:
        out[...] = h


def _tc_layer(z0, z1, tp, dv, b, g, be, wnt):
    hid_spec = pl.BlockSpec((_BLK, HID), lambda i: (i, 0))
    vec_spec = pl.BlockSpec((1, HID), lambda i: (0, 0))
    in_specs = [hid_spec, hid_spec, hid_spec,
                pl.BlockSpec((_BLK, 1), lambda i: (i, 0)),
                vec_spec, vec_spec, vec_spec]
    args = [z0, z1, tp, dv, b, g, be]
    if wnt is not None:
        body = _layer_body
        in_specs.append(pl.BlockSpec((HID, HID), lambda i: (0, 0)))
        args.append(wnt)
    else:
        def body(z0, z1, tp, dv, b, g, be, out):
            _layer_body(z0, z1, tp, dv, b, g, be, None, out)
    return pl.pallas_call(
        body,
        grid=(NP // _BLK,),
        in_specs=in_specs,
        out_specs=hid_spec,
        out_shape=jax.ShapeDtypeStruct((NP, HID), jnp.float32),
    )(*args)


_EBLK = 4096


def _head_body(gs, gd, cwt, cb, out):
    out[...] = (jnp.dot(gs[...] * gd[...], cwt[...],
                        preferred_element_type=jnp.float32) + cb[...])


def _tc_head(gs, gd, cwt, cb):
    return pl.pallas_call(
        _head_body,
        grid=(E_PAD // _EBLK,),
        in_specs=[
            pl.BlockSpec((_EBLK, HID), lambda i: (i, 0)),
            pl.BlockSpec((_EBLK, HID), lambda i: (i, 0)),
            pl.BlockSpec((HID, NCLS), lambda i: (0, 0)),
            pl.BlockSpec((1, NCLS), lambda i: (0, 0)),
        ],
        out_specs=pl.BlockSpec((_EBLK, NCLS), lambda i: (i, 0)),
        out_shape=jax.ShapeDtypeStruct((E_PAD, NCLS), jnp.float32),
    )(gs, gd, cwt, cb)


def kernel(node_feat, edge_index, fc1_w, fc1_b, fc2_w, fc2_b, fc3_w, fc3_b,
           ln1_g, ln1_b, ln2_g, ln2_b, ln3_g, ln3_b, cls_w, cls_b):
    ei = edge_index.astype(jnp.int32)
    src = jnp.concatenate(
        [ei[0], jnp.zeros((E_PAD - E,), jnp.int32)]).reshape(NW, NCH, CHUNK)
    dst = jnp.concatenate(
        [ei[1], jnp.full((E_PAD - E,), N, jnp.int32)]).reshape(NW, NCH, CHUNK)

    nf_p = jnp.pad(node_feat, ((0, NP - N), (0, 0)))
    zeros_deg = jnp.zeros((NP, DEG_W), jnp.float32)
    ones_deg = jnp.ones((CHUNK, DEG_W), jnp.float32)
    zeros_hid = jnp.zeros((NP, HID), jnp.float32)

    degp = _sc_degree(dst, ones_deg, zeros_deg)
    d0 = degp[0, :, :1]
    d1 = degp[1, :, :1]

    t1p, dv = _tc_prep(nf_p, fc1_w.T, d0, d1)
    z = _sc_spmm(t1p, src, dst, zeros_hid)
    t2p = _tc_layer(z[0], z[1], t1p, dv,
                    fc1_b.reshape(1, HID), ln1_g.reshape(1, HID),
                    ln1_b.reshape(1, HID), fc2_w.T)
    z = _sc_spmm(t2p, src, dst, zeros_hid)
    t3p = _tc_layer(z[0], z[1], t2p, dv,
                    fc2_b.reshape(1, HID), ln2_g.reshape(1, HID),
                    ln2_b.reshape(1, HID), fc3_w.T)
    z = _sc_spmm(t3p, src, dst, zeros_hid)
    h3 = _tc_layer(z[0], z[1], t3p, dv,
                   fc3_b.reshape(1, HID), ln3_g.reshape(1, HID),
                   ln3_b.reshape(1, HID), None)

    gs, gd = _sc_edge_gather(h3, src, dst)
    logits = _tc_head(gs.reshape(E_PAD, HID), gd.reshape(E_PAD, HID),
                      cls_w.T, cls_b.reshape(1, NCLS))
    return logits[:E]


# speedup vs baseline: 1.5971x; 1.1236x over previous
import functools

import jax
import jax.numpy as jnp
from jax import lax
from jax.experimental import pallas as pl
from jax.experimental.pallas import tpu as pltpu
from jax.experimental.pallas import tpu_sc as plsc

N = 10000
E = 320000
IN_DIM = 128
HID = 32
NCLS = 2

NC = 2
NS = 16
NW = NC * NS
CHUNK = 128
GK = 4
NCH = 80
E_PAD = NW * NCH * CHUNK
NP = 10112
RPT = NP // NS
DEG_W = 16
DEG_KB = 8

_mesh = plsc.VectorSubcoreMesh(core_axis_name="c", subcore_axis_name="s")
_sc_params = pltpu.CompilerParams(use_tc_tiling_on_sc=False)


def _worker_id():
    return lax.axis_index("s") * NC + lax.axis_index("c")


@functools.partial(
    pl.kernel,
    out_type=jax.ShapeDtypeStruct((NC, NP, DEG_W), jnp.float32),
    mesh=_mesh,
    compiler_params=_sc_params,
    scratch_types=[
        pltpu.VMEM_SHARED((NP, DEG_W), jnp.float32),
        pltpu.VMEM((CHUNK, DEG_W), jnp.float32),
        pltpu.VMEM((NCH, CHUNK), jnp.int32),
        pltpu.SemaphoreType.DMA,
    ],
)
def _sc_degree(dst3, ones_hbm, zeros_hbm, out, acc, ones_v, idx_d, sem):
    cid = lax.axis_index("c")
    sid = lax.axis_index("s")
    wid = _worker_id()
    base = sid * RPT
    pltpu.sync_copy(dst3.at[wid], idx_d)
    pltpu.sync_copy(ones_hbm, ones_v)
    pltpu.sync_copy(zeros_hbm.at[pl.ds(base, RPT)], acc.at[pl.ds(base, RPT)])
    plsc.subcore_barrier()

    def body(j0, carry):
        descs = [
            pltpu.async_copy(ones_v, acc.at[idx_d.at[j0 * DEG_KB + b]], sem,
                             add=True)
            for b in range(DEG_KB)
        ]
        for d in descs:
            d.wait()
        return carry

    lax.fori_loop(0, NCH // DEG_KB, body, 0)
    plsc.subcore_barrier()
    pltpu.sync_copy(acc.at[pl.ds(base, RPT)], out.at[cid, pl.ds(base, RPT)])


@functools.partial(
    pl.kernel,
    out_type=jax.ShapeDtypeStruct((NC, NP, HID), jnp.float32),
    mesh=_mesh,
    compiler_params=_sc_params,
    scratch_types=[
        pltpu.VMEM_SHARED((NP, HID), jnp.float32),
        [pltpu.VMEM((CHUNK, HID), jnp.float32) for _ in range(2 * GK)],
        pltpu.VMEM((NCH, CHUNK), jnp.int32),
        pltpu.VMEM((NCH, CHUNK), jnp.int32),
        pltpu.SemaphoreType.DMA,
        pltpu.SemaphoreType.DMA,
    ],
)
def _sc_spmm(xp, src3, dst3, zeros_hbm, out, acc, rows, idx_s, idx_d,
             gsem, ssem):
    cid = lax.axis_index("c")
    sid = lax.axis_index("s")
    wid = _worker_id()
    base = sid * RPT
    pltpu.sync_copy(src3.at[wid], idx_s)
    pltpu.sync_copy(dst3.at[wid], idx_d)
    pltpu.sync_copy(zeros_hbm.at[pl.ds(base, RPT)], acc.at[pl.ds(base, RPT)])
    plsc.subcore_barrier()

    def body(j0, carry):
        c0 = j0 * 2 * GK
        ga = [pltpu.async_copy(xp.at[idx_s.at[c0 + b]], rows[b], gsem)
              for b in range(GK)]
        for d in ga:
            d.wait()
        sa = [pltpu.async_copy(rows[b], acc.at[idx_d.at[c0 + b]], ssem,
                               add=True) for b in range(GK)]
        gb = [pltpu.async_copy(xp.at[idx_s.at[c0 + GK + b]], rows[GK + b],
                               gsem) for b in range(GK)]
        for d in gb:
            d.wait()
        sb = [pltpu.async_copy(rows[GK + b], acc.at[idx_d.at[c0 + GK + b]],
                               ssem, add=True) for b in range(GK)]
        for d in sa + sb:
            d.wait()
        return carry

    lax.fori_loop(0, NCH // (2 * GK), body, 0)
    plsc.subcore_barrier()
    pltpu.sync_copy(acc.at[pl.ds(base, RPT)], out.at[cid, pl.ds(base, RPT)])


@functools.partial(
    pl.kernel,
    out_type=(
        jax.ShapeDtypeStruct((E_PAD, HID), jnp.float32),
        jax.ShapeDtypeStruct((E_PAD, HID), jnp.float32),
    ),
    mesh=_mesh,
    compiler_params=_sc_params,
    scratch_types=[
        [pltpu.VMEM((CHUNK, HID), jnp.float32) for _ in range(2 * GK)],
        [pltpu.VMEM((CHUNK, HID), jnp.float32) for _ in range(2 * GK)],
        pltpu.VMEM((NCH, CHUNK), jnp.int32),
        pltpu.VMEM((NCH, CHUNK), jnp.int32),
        pltpu.SemaphoreType.DMA,
        pltpu.SemaphoreType.DMA,
    ],
)
def _sc_edge_gather(h3, src3, dst3, gs, gd, rows_s, rows_d, idx_s, idx_d,
                    gsem, wsem):
    wid = _worker_id()
    woff = wid * (NCH * CHUNK)
    pltpu.sync_copy(src3.at[wid], idx_s)
    pltpu.sync_copy(dst3.at[wid], idx_d)

    def grp_gather(c0, lo):
        descs = []
        for b in range(GK):
            descs.append(pltpu.async_copy(h3.at[idx_s.at[c0 + b]],
                                          rows_s[lo + b], gsem))
            descs.append(pltpu.async_copy(h3.at[idx_d.at[c0 + b]],
                                          rows_d[lo + b], gsem))
        return descs

    def grp_write(c0, lo):
        descs = []
        for b in range(GK):
            j = c0 + b
            descs.append(pltpu.async_copy(
                rows_s[lo + b], gs.at[pl.ds(woff + j * CHUNK, CHUNK)], wsem))
            descs.append(pltpu.async_copy(
                rows_d[lo + b], gd.at[pl.ds(woff + j * CHUNK, CHUNK)], wsem))
        return descs

    def body(j0, carry):
        c0 = j0 * 2 * GK
        ga = grp_gather(c0, 0)
        for d in ga:
            d.wait()
        wa = grp_write(c0, 0)
        gb = grp_gather(c0 + GK, GK)
        for d in gb:
            d.wait()
        wb = grp_write(c0 + GK, GK)
        for d in wa + wb:
            d.wait()
        return carry

    lax.fori_loop(0, NCH // (2 * GK), body, 0)


_BLK = 2528
_EPS = 1e-5


def _prep_body(nf, w1t, degp, tp, dv):
    deg = degp[0][:, :1] + degp[1][:, :1] + 1.0
    di = lax.rsqrt(deg)
    t = jnp.dot(nf[...], w1t[...], preferred_element_type=jnp.float32)
    tp[...] = di * t
    dv[...] = di


def _tc_prep(nf_p, w1t, degp):
    return pl.pallas_call(
        _prep_body,
        grid=(NP // _BLK,),
        in_specs=[
            pl.BlockSpec((_BLK, IN_DIM), lambda i: (i, 0)),
            pl.BlockSpec((IN_DIM, HID), lambda i: (0, 0)),
            pl.BlockSpec((NC, _BLK, DEG_W), lambda i: (0, i, 0)),
        ],
        out_specs=[
            pl.BlockSpec((_BLK, HID), lambda i: (i, 0)),
            pl.BlockSpec((_BLK, 1), lambda i: (i, 0)),
        ],
        out_shape=[
            jax.ShapeDtypeStruct((NP, HID), jnp.float32),
            jax.ShapeDtypeStruct((NP, 1), jnp.float32),
        ],
    )(nf_p, w1t, degp)


def _layer_body(z, tp, dv, b, g, be, wnt, out):
    di = dv[...]
    s = di * (z[0] + z[1] + tp[...]) + b[...]
    mu = jnp.mean(s, axis=-1, keepdims=True)
    var = jnp.mean((s - mu) ** 2, axis=-1, keepdims=True)
    h = jnp.maximum((s - mu) * lax.rsqrt(var + _EPS) * g[...] + be[...], 0.0)
    if wnt is not None:
        out[...] = di * jnp.dot(h, wnt[...], preferred_element_type=jnp.float32)
    else:
        out[...] = h


def _tc_layer(z, tp, dv, b, g, be, wnt):
    hid_spec = pl.BlockSpec((_BLK, HID), lambda i: (i, 0))
    vec_spec = pl.BlockSpec((1, HID), lambda i: (0, 0))
    in_specs = [pl.BlockSpec((NC, _BLK, HID), lambda i: (0, i, 0)),
                hid_spec,
                pl.BlockSpec((_BLK, 1), lambda i: (i, 0)),
                vec_spec, vec_spec, vec_spec]
    args = [z, tp, dv, b, g, be]
    if wnt is not None:
        body = _layer_body
        in_specs.append(pl.BlockSpec((HID, HID), lambda i: (0, 0)))
        args.append(wnt)
    else:
        def body(z, tp, dv, b, g, be, out):
            _layer_body(z, tp, dv, b, g, be, None, out)
    return pl.pallas_call(
        body,
        grid=(NP // _BLK,),
        in_specs=in_specs,
        out_specs=hid_spec,
        out_shape=jax.ShapeDtypeStruct((NP, HID), jnp.float32),
    )(*args)


_EBLK = 2560


def _head_body(gs, gd, cwt, cb, out):
    out[...] = (jnp.dot(gs[...] * gd[...], cwt[...],
                        preferred_element_type=jnp.float32) + cb[...])


def _tc_head(gs, gd, cwt, cb):
    return pl.pallas_call(
        _head_body,
        grid=(E // _EBLK,),
        in_specs=[
            pl.BlockSpec((_EBLK, HID), lambda i: (i, 0)),
            pl.BlockSpec((_EBLK, HID), lambda i: (i, 0)),
            pl.BlockSpec((HID, NCLS), lambda i: (0, 0)),
            pl.BlockSpec((1, NCLS), lambda i: (0, 0)),
        ],
        out_specs=pl.BlockSpec((_EBLK, NCLS), lambda i: (i, 0)),
        out_shape=jax.ShapeDtypeStruct((E, NCLS), jnp.float32),
    )(gs, gd, cwt, cb)


def kernel(node_feat, edge_index, fc1_w, fc1_b, fc2_w, fc2_b, fc3_w, fc3_b,
           ln1_g, ln1_b, ln2_g, ln2_b, ln3_g, ln3_b, cls_w, cls_b):
    ei = edge_index.astype(jnp.int32)
    src = jnp.concatenate(
        [ei[0], jnp.zeros((E_PAD - E,), jnp.int32)]).reshape(NW, NCH, CHUNK)
    dst = jnp.concatenate(
        [ei[1], jnp.full((E_PAD - E,), N, jnp.int32)]).reshape(NW, NCH, CHUNK)

    nf_p = jnp.pad(node_feat, ((0, NP - N), (0, 0)))
    zeros_deg = jnp.zeros((NP, DEG_W), jnp.float32)
    ones_deg = jnp.ones((CHUNK, DEG_W), jnp.float32)
    zeros_hid = jnp.zeros((NP, HID), jnp.float32)

    degp = _sc_degree(dst, ones_deg, zeros_deg)

    t1p, dv = _tc_prep(nf_p, fc1_w.T, degp)
    z = _sc_spmm(t1p, src, dst, zeros_hid)
    t2p = _tc_layer(z, t1p, dv,
                    fc1_b.reshape(1, HID), ln1_g.reshape(1, HID),
                    ln1_b.reshape(1, HID), fc2_w.T)
    z = _sc_spmm(t2p, src, dst, zeros_hid)
    t3p = _tc_layer(z, t2p, dv,
                    fc2_b.reshape(1, HID), ln2_g.reshape(1, HID),
                    ln2_b.reshape(1, HID), fc3_w.T)
    z = _sc_spmm(t3p, src, dst, zeros_hid)
    h3 = _tc_layer(z, t3p, dv,
                   fc3_b.reshape(1, HID), ln3_g.reshape(1, HID),
                   ln3_b.reshape(1, HID), None)

    gs, gd = _sc_edge_gather(h3, src, dst)
    return _tc_head(gs, gd, cls_w.T, cls_b.reshape(1, NCLS))
